# baseline (speedup 1.0000x reference)
import jax
import jax.numpy as jnp
from jax import lax
from jax.experimental import pallas as pl
from jax.experimental.pallas import tpu as pltpu
from jax.experimental.pallas import tpu_sc as plsc

_VOCAB = 32000
_NC, _NS, _L = 2, 16, 16


def _body(ids_hbm, anchor_hbm, out_hbm, ids_v, anc_v, zbuf, sem):
    b = lax.axis_index("c") * _NS + lax.axis_index("s")
    s = out_hbm.shape[1]

    pltpu.sync_copy(ids_hbm.at[b, pl.ds(s - _L, _L)], ids_v)
    pltpu.sync_copy(anchor_hbm, anc_v.at[pl.ds(0, 1)])

    z16 = jnp.zeros((_L,), jnp.float32)

    def _fill(i, carry):
        zbuf[pl.ds(i * _L, _L)] = z16
        return carry

    lax.fori_loop(0, _VOCAB // _L, _fill, 0)

    def _fire(j, carry):
        pltpu.make_async_copy(zbuf, out_hbm.at[b, j, :], sem).start()
        return carry

    lax.fori_loop(0, s - 1, _fire, 0)

    tok = (ids_v[...][_L - 1] + 1) % (_VOCAB - 1) + 1
    val = 10.0 + anc_v[...][0]
    off = (tok // _L) * _L
    vec = jnp.where(lax.iota(jnp.int32, _L) == tok - off, val, 0.0)

    def _drain(j, carry):
        pltpu.make_async_copy(zbuf, out_hbm.at[b, 0, :], sem).wait()
        return carry

    lax.fori_loop(0, s - 1, _drain, 0)

    zbuf[pl.ds(off, _L)] = vec
    pltpu.make_async_copy(zbuf, out_hbm.at[b, s - 1, :], sem).start()
    pltpu.make_async_copy(zbuf, out_hbm.at[b, s - 1, :], sem).wait()


def kernel(input_ids, anchor):
    batch, seq_len = input_ids.shape
    mesh = plsc.VectorSubcoreMesh(
        core_axis_name="c", subcore_axis_name="s",
        num_cores=_NC, num_subcores=_NS,
    )
    k = pl.kernel(
        _body,
        out_type=jax.ShapeDtypeStruct((batch, seq_len, _VOCAB), jnp.float32),
        mesh=mesh,
        scratch_types=[
            pltpu.VMEM((_L,), jnp.int32),
            pltpu.VMEM((_L,), jnp.float32),
            pltpu.VMEM((_VOCAB,), jnp.float32),
            pltpu.SemaphoreType.DMA,
        ],
    )
    return k(input_ids, anchor)


# speedup vs baseline: 1.2912x; 1.2912x over previous
import jax
import jax.numpy as jnp
from jax.experimental import pallas as pl
from jax.experimental.pallas import tpu as pltpu

_VOCAB = 32000
_SSPLIT = 2


def _body(ids_ref, anchor_ref, out_ref):
    b = pl.program_id(0)
    j = pl.program_id(1)
    blk_s = out_ref.shape[1]
    out_ref[...] = jnp.zeros(out_ref.shape, jnp.float32)

    @pl.when(j == pl.num_programs(1) - 1)
    def _last_row():
        s_total = blk_s * pl.num_programs(1)
        tok = (ids_ref[b, s_total - 1] + 1) % (_VOCAB - 1) + 1
        val = 10.0 + anchor_ref[0]
        col = jax.lax.broadcasted_iota(jnp.int32, (1, _VOCAB), 1)
        out_ref[:, blk_s - 1, :] = jnp.where(col == tok, val, 0.0)


def kernel(input_ids, anchor):
    batch, seq_len = input_ids.shape
    grid_spec = pltpu.PrefetchScalarGridSpec(
        num_scalar_prefetch=2,
        grid=(batch, _SSPLIT),
        in_specs=[],
        out_specs=pl.BlockSpec(
            (1, seq_len // _SSPLIT, _VOCAB),
            lambda b, j, ids, anc: (b, j, 0),
        ),
    )
    return pl.pallas_call(
        _body,
        grid_spec=grid_spec,
        out_shape=jax.ShapeDtypeStruct((batch, seq_len, _VOCAB), jnp.float32),
    )(input_ids, anchor)


# speedup vs baseline: 1.6319x; 1.2639x over previous
import jax
import jax.numpy as jnp
from jax.experimental import pallas as pl
from jax.experimental.pallas import tpu as pltpu

_VOCAB = 32000


_BB = 2


def _body(ids_ref, anchor_ref, out_ref):
    g = pl.program_id(0)
    s = out_ref.shape[1]
    val = 10.0 + anchor_ref[0]
    out_ref[...] = jnp.zeros(out_ref.shape, jnp.float32)
    col = jax.lax.broadcasted_iota(jnp.int32, (1, _VOCAB), 1)
    for i in range(_BB):
        tok = (ids_ref[g * _BB + i, s - 1] + 1) % (_VOCAB - 1) + 1
        out_ref[i, pl.ds(s - 1, 1), :] = jnp.where(col == tok, val, 0.0)


def kernel(input_ids, anchor):
    batch, seq_len = input_ids.shape
    grid_spec = pltpu.PrefetchScalarGridSpec(
        num_scalar_prefetch=2,
        grid=(batch // _BB,),
        in_specs=[],
        out_specs=pl.BlockSpec(
            (_BB, seq_len, _VOCAB), lambda g, ids, anc: (g, 0, 0)
        ),
    )
    return pl.pallas_call(
        _body,
        grid_spec=grid_spec,
        out_shape=jax.ShapeDtypeStruct((batch, seq_len, _VOCAB), jnp.float32),
    )(input_ids, anchor)


# speedup vs baseline: 1.6557x; 1.0146x over previous
import jax
import jax.numpy as jnp
from jax.experimental import pallas as pl
from jax.experimental.pallas import tpu as pltpu

_VOCAB = 32000


def _body(ids_ref, anchor_ref, out_ref):
    b = pl.program_id(0)
    s = out_ref.shape[1]
    tok = (ids_ref[b, s - 1] + 1) % (_VOCAB - 1) + 1
    val = 10.0 + anchor_ref[0]
    out_ref[...] = jnp.zeros(out_ref.shape, jnp.float32)
    col = jax.lax.broadcasted_iota(jnp.int32, (1, _VOCAB), 1)
    out_ref[:, s - 1, :] = jnp.where(col == tok, val, 0.0)


def kernel(input_ids, anchor):
    batch, seq_len = input_ids.shape
    grid_spec = pltpu.PrefetchScalarGridSpec(
        num_scalar_prefetch=2,
        grid=(batch,),
        in_specs=[],
        out_specs=pl.BlockSpec(
            (1, seq_len, _VOCAB), lambda b, ids, anc: (b, 0, 0)
        ),
    )
    return pl.pallas_call(
        _body,
        grid_spec=grid_spec,
        out_shape=jax.ShapeDtypeStruct((batch, seq_len, _VOCAB), jnp.float32),
        compiler_params=pltpu.CompilerParams(
            dimension_semantics=("parallel",),
        ),
    )(input_ids, anchor)
